# baseline (speedup 1.0000x reference)
import functools

import jax
import jax.numpy as jnp
from jax import lax
from jax.experimental import pallas as pl
from jax.experimental.pallas import tpu as pltpu
from jax.experimental.pallas import tpu_sc as plsc

N = 10000
E = 320000
D = 128

NC = 2
NS = 16
NW = NC * NS
EPW = E // NW
K = 16
C = EPW // K
N_PAD = 10240
RPT = N_PAD // NS
FR = 64

_MESH = plsc.VectorSubcoreMesh(core_axis_name="c", subcore_axis_name="s")

_SELU_L = 1.0507009873554804934193349852946
_SELU_A = 1.6732632423543772848170429916717


@functools.partial(
    pl.kernel,
    out_type=jax.ShapeDtypeStruct((NW * 4 * N,), jnp.float32),
    mesh=_MESH,
    compiler_params=pltpu.CompilerParams(needs_layout_passes=False),
    scratch_types=[
        pltpu.VMEM((EPW,), jnp.int32),
        pltpu.VMEM((EPW,), jnp.int32),
        pltpu.VMEM((EPW,), jnp.float32),
        pltpu.VMEM((4 * N,), jnp.float32),
    ],
)
def _deg_kernel(src_hbm, dst_hbm, w_hbm, out_hbm, src_v, dst_v, w_v, acc_v):
    cid = lax.axis_index("c")
    sid = lax.axis_index("s")
    wid = sid * NC + cid

    pltpu.sync_copy(src_hbm.at[pl.ds(wid * EPW, EPW)], src_v)
    pltpu.sync_copy(dst_hbm.at[pl.ds(wid * EPW, EPW)], dst_v)
    pltpu.sync_copy(w_hbm.at[pl.ds(wid * EPW, EPW)], w_v)

    def zero_body(i, _):
        acc_v[pl.ds(i * K, K)] = jnp.zeros((K,), jnp.float32)
        return 0

    lax.fori_loop(0, (4 * N) // K, zero_body, 0)

    ones = jnp.ones((K,), jnp.float32)

    def edge_body(g, _):
        s16 = src_v[pl.ds(g * K, K)]
        d16 = dst_v[pl.ds(g * K, K)]
        w16 = w_v[pl.ds(g * K, K)]
        plsc.addupdate_scatter(acc_v, [s16], w16)
        plsc.addupdate_scatter(acc_v, [d16 + N], w16)
        plsc.addupdate_scatter(acc_v, [s16 + 2 * N], ones)
        plsc.addupdate_scatter(acc_v, [d16 + 3 * N], ones)
        return 0

    lax.fori_loop(0, C, edge_body, 0)

    pltpu.sync_copy(acc_v, out_hbm.at[pl.ds(wid * 4 * N, 4 * N)])


CK = 80
CC = EPW // CK
QG = CK // K
RPF = 640


def _spmm_scale(rows_b, w_v, g):
    for q in range(QG):
        w16 = w_v[pl.ds(g * CK + q * K, K)]
        for k in range(K):
            wb = w16.at[jnp.full((K,), k, jnp.int32)].get(
                mode="promise_in_bounds")
            e = q * K + k
            for r in range(D // K):
                sl = pl.ds(r * K, K)
                rows_b[e, sl] = rows_b[e, sl] * wb


def _spmm_scatter(rows_b, dst_v, acc_sh, g, sem):
    descs = []
    for q in range(QG):
        d16 = dst_v[pl.ds(g * CK + q * K, K)]
        descs.append(pltpu.async_copy(
            rows_b.at[pl.ds(q * K, K)], acc_sh.at[d16], sem, add=True))
    return descs


NB = 3


@functools.partial(
    pl.kernel,
    out_type=jax.ShapeDtypeStruct((NC, N, D), jnp.float32),
    mesh=_MESH,
    compiler_params=pltpu.CompilerParams(needs_layout_passes=False),
    scratch_types=[
        [pltpu.VMEM((CK,), jnp.int32) for _ in range(NB)],
        [pltpu.VMEM((CK,), jnp.int32) for _ in range(NB)],
        pltpu.VMEM((EPW,), jnp.float32),
        [pltpu.VMEM((CK, D), jnp.float32) for _ in range(NB)],
        pltpu.VMEM_SHARED((N, D), jnp.float32),
        [pltpu.SemaphoreType.DMA for _ in range(NB)],
        [pltpu.SemaphoreType.DMA for _ in range(NB)],
        [pltpu.SemaphoreType.DMA for _ in range(NB)],
    ],
)
def _spmm_kernel(feat_hbm, src_hbm, dst_hbm, w_hbm, out_hbm,
                 srcs, dsts, w_v, rows, acc_sh, gsems, msems, tsems):
    cid = lax.axis_index("c")
    sid = lax.axis_index("s")
    wid = sid * NC + cid
    ebase = wid * EPW

    pltpu.sync_copy(w_hbm.at[pl.ds(ebase, EPW)], w_v)

    def zbuf_row(i, _):
        for r in range(D // K):
            rows[0][i, pl.ds(r * K, K)] = jnp.zeros((K,), jnp.float32)
        return 0

    lax.fori_loop(0, CK, zbuf_row, 0)
    nflush = RPF // CK

    @pl.when(sid < NS - 1)
    def _():
        for j in range(nflush):
            pltpu.sync_copy(rows[0],
                            acc_sh.at[pl.ds(sid * RPF + j * CK, CK)])

    @pl.when(sid == NS - 1)
    def _():
        for j in range((N - (NS - 1) * RPF) // CK):
            pltpu.sync_copy(rows[0],
                            acc_sh.at[pl.ds((NS - 1) * RPF + j * CK, CK)])

    plsc.subcore_barrier()

    for s in range(NB):
        pltpu.sync_copy(src_hbm.at[pl.ds(ebase + s * CK, CK)], srcs[s])
        pltpu.sync_copy(dst_hbm.at[pl.ds(ebase + s * CK, CK)], dsts[s])
    for s in range(NB):
        pltpu.async_copy(feat_hbm.at[srcs[s]], rows[s], gsems[s])

    def _drain_scatter(s, buf):
        d16 = dsts[s][pl.ds(0, K)]
        for q in range(QG):
            pltpu.make_async_copy(buf.at[pl.ds(q * K, K)],
                                  acc_sh.at[d16], tsems[s]).wait()

    def chunk_body(g, _):
        for s in range(NB):
            @pl.when(g % NB == s)
            def _(s=s):
                buf = rows[s]
                pltpu.make_async_copy(feat_hbm.at[srcs[s]], buf,
                                      gsems[s]).wait()
                _spmm_scale(buf, w_v, g)
                for q in range(QG):
                    d16 = dsts[s][pl.ds(q * K, K)]
                    pltpu.async_copy(buf.at[pl.ds(q * K, K)],
                                     acc_sh.at[d16], tsems[s], add=True)

                s2 = (s + NB - 1) % NB

                @pl.when(g >= 1)
                def _():
                    _drain_scatter(s2, rows[s2])

                @pl.when(g + NB < CC)
                def _():
                    pltpu.async_copy(
                        src_hbm.at[pl.ds(ebase + (g + NB) * CK, CK)],
                        srcs[s], msems[s])
                    pltpu.async_copy(
                        dst_hbm.at[pl.ds(ebase + (g + NB) * CK, CK)],
                        dsts[s], msems[s])

                @pl.when((g >= 1) & (g + 2 < CC))
                def _():
                    pltpu.make_async_copy(
                        src_hbm.at[pl.ds(ebase + (g + 2) * CK, CK)],
                        srcs[s2], msems[s2]).wait()
                    pltpu.make_async_copy(
                        dst_hbm.at[pl.ds(ebase + (g + 2) * CK, CK)],
                        dsts[s2], msems[s2]).wait()
                    pltpu.async_copy(feat_hbm.at[srcs[s2]], rows[s2],
                                     gsems[s2])

        return 0

    lax.fori_loop(0, CC, chunk_body, 0)
    s_last = (CC - 1) % NB
    _drain_scatter(s_last, rows[s_last])
    plsc.subcore_barrier()

    @pl.when(sid < NS - 1)
    def _():
        for j in range(nflush):
            rs = sid * RPF + j * CK
            pltpu.sync_copy(acc_sh.at[pl.ds(rs, CK)], rows[0])
            pltpu.sync_copy(rows[0], out_hbm.at[cid, pl.ds(rs, CK)])

    @pl.when(sid == NS - 1)
    def _():
        for j in range((N - (NS - 1) * RPF) // CK):
            rs = (NS - 1) * RPF + j * CK
            pltpu.sync_copy(acc_sh.at[pl.ds(rs, CK)], rows[0])
            pltpu.sync_copy(rows[0], out_hbm.at[cid, pl.ds(rs, CK)])


def _prep_body(dp_ref, x_ref, alpha_ref, beta_ref, feat_ref):
    s = jnp.sum(dp_ref[...], axis=0)
    wdeg_src = s[0]
    wdeg_dst = s[1]
    out_deg = jnp.maximum(s[2], 1.0)
    in_deg = jnp.maximum(s[3], 1.0)
    alpha = lax.rsqrt(wdeg_src * out_deg)
    beta = lax.rsqrt(wdeg_dst * in_deg)
    alpha2 = alpha[:, None]
    beta2 = beta[:, None]
    alpha_ref[...] = alpha2
    beta_ref[...] = beta2
    feat_ref[...] = x_ref[...] * alpha2


def _prep(deg_parts, x):
    return pl.pallas_call(
        _prep_body,
        out_shape=(
            jax.ShapeDtypeStruct((N, 1), jnp.float32),
            jax.ShapeDtypeStruct((N, 1), jnp.float32),
            jax.ShapeDtypeStruct((N, D), jnp.float32),
        ),
    )(deg_parts, x)


_RB = 1000


def _stage_body(last, p0_ref, p1_ref, beta_ref, W_ref, b_ref, alpha_ref,
                *out_refs):
    agg = (p0_ref[...] + p1_ref[...]) * beta_ref[...]
    z = jnp.dot(agg, W_ref[...], preferred_element_type=jnp.float32)
    z = z + b_ref[...]
    zn = jnp.minimum(z, 0.0)
    poly = zn * (1.0 + zn * (0.5 + zn * (1.0 / 6.0 + zn * (1.0 / 24.0))))
    em1 = jnp.where(zn > -0.05, poly, jnp.exp(zn) - 1.0)
    h = _SELU_L * jnp.where(z > 0, z, _SELU_A * em1)
    if last:
        out_refs[0][...] = h
    else:
        out_refs[0][...] = h * alpha_ref[...]


def _stage(p0, p1, beta, W, b, alpha, last):
    grid = N // _RB
    return pl.pallas_call(
        functools.partial(_stage_body, last),
        grid=(grid,),
        in_specs=[
            pl.BlockSpec((_RB, D), lambda i: (i, 0)),
            pl.BlockSpec((_RB, D), lambda i: (i, 0)),
            pl.BlockSpec((_RB, 1), lambda i: (i, 0)),
            pl.BlockSpec((D, D), lambda i: (0, 0)),
            pl.BlockSpec((1, D), lambda i: (0, 0)),
            pl.BlockSpec((_RB, 1), lambda i: (i, 0)),
        ],
        out_specs=pl.BlockSpec((_RB, D), lambda i: (i, 0)),
        out_shape=jax.ShapeDtypeStruct((N, D), jnp.float32),
    )(p0, p1, beta, W, b, alpha)


def kernel(x, edge_index, edge_weight, W1_0, b1_0, W2_0, b2_0,
           W1_1, b1_1, W2_1, b2_1):
    src_r = edge_index[0]
    dst_r = edge_index[1]
    w_r = edge_weight

    deg_parts = _deg_kernel(src_r, dst_r, w_r).reshape(NW, 4, N)
    alpha, beta, feat = _prep(deg_parts, x)

    weights = ((W1_0, b1_0), (W2_0, b2_0), (W1_1, b1_1), (W2_1, b2_1))
    for i, (W, b) in enumerate(weights):
        parts = _spmm_kernel(feat, src_r, dst_r, w_r)
        feat = _stage(parts[0], parts[1], beta, W, b.reshape(1, D), alpha,
                      last=(i == 3))
    return feat


# speedup vs baseline: 1.2736x; 1.2736x over previous
import functools

import jax
import jax.numpy as jnp
from jax import lax
from jax.experimental import pallas as pl
from jax.experimental.pallas import tpu as pltpu
from jax.experimental.pallas import tpu_sc as plsc

N = 10000
E = 320000
D = 128

NC = 2
NS = 16
NW = NC * NS
EPW = E // NW
K = 16
C = EPW // K
N_PAD = 10240
RPT = N_PAD // NS
FR = 64

_MESH = plsc.VectorSubcoreMesh(core_axis_name="c", subcore_axis_name="s")

_SELU_L = 1.0507009873554804934193349852946
_SELU_A = 1.6732632423543772848170429916717


@functools.partial(
    pl.kernel,
    out_type=jax.ShapeDtypeStruct((NW * 4 * N,), jnp.float32),
    mesh=_MESH,
    compiler_params=pltpu.CompilerParams(needs_layout_passes=False),
    scratch_types=[
        pltpu.VMEM((EPW,), jnp.int32),
        pltpu.VMEM((EPW,), jnp.int32),
        pltpu.VMEM((EPW,), jnp.float32),
        pltpu.VMEM((4 * N,), jnp.float32),
    ],
)
def _deg_kernel(src_hbm, dst_hbm, w_hbm, out_hbm, src_v, dst_v, w_v, acc_v):
    cid = lax.axis_index("c")
    sid = lax.axis_index("s")
    wid = sid * NC + cid

    pltpu.sync_copy(src_hbm.at[pl.ds(wid * EPW, EPW)], src_v)
    pltpu.sync_copy(dst_hbm.at[pl.ds(wid * EPW, EPW)], dst_v)
    pltpu.sync_copy(w_hbm.at[pl.ds(wid * EPW, EPW)], w_v)

    def zero_body(i, _):
        acc_v[pl.ds(i * K, K)] = jnp.zeros((K,), jnp.float32)
        return 0

    lax.fori_loop(0, (4 * N) // K, zero_body, 0)

    ones = jnp.ones((K,), jnp.float32)

    def edge_body(g, _):
        s16 = src_v[pl.ds(g * K, K)]
        d16 = dst_v[pl.ds(g * K, K)]
        w16 = w_v[pl.ds(g * K, K)]
        plsc.addupdate_scatter(acc_v, [s16], w16)
        plsc.addupdate_scatter(acc_v, [d16 + N], w16)
        plsc.addupdate_scatter(acc_v, [s16 + 2 * N], ones)
        plsc.addupdate_scatter(acc_v, [d16 + 3 * N], ones)
        return 0

    lax.fori_loop(0, C, edge_body, 0)

    pltpu.sync_copy(acc_v, out_hbm.at[pl.ds(wid * 4 * N, 4 * N)])


CK = 80
CC = EPW // CK
QG = CK // K
RPF = 640


def _spmm_scale(rows_b, w_v, g):
    for q in range(QG):
        w16 = w_v[pl.ds(g * CK + q * K, K)]
        for k in range(K):
            wb = w16.at[jnp.full((K,), k, jnp.int32)].get(
                mode="promise_in_bounds")
            e = q * K + k
            for r in range(D // K):
                sl = pl.ds(r * K, K)
                rows_b[e, sl] = rows_b[e, sl] * wb


def _spmm_scatter(rows_b, dst_v, acc_sh, g, sem):
    descs = []
    for q in range(QG):
        d16 = dst_v[pl.ds(g * CK + q * K, K)]
        descs.append(pltpu.async_copy(
            rows_b.at[pl.ds(q * K, K)], acc_sh.at[d16], sem, add=True))
    return descs


@functools.partial(
    pl.kernel,
    out_type=jax.ShapeDtypeStruct((NC, N, D), jnp.float32),
    mesh=_MESH,
    compiler_params=pltpu.CompilerParams(needs_layout_passes=False),
    scratch_types=[
        pltpu.VMEM((EPW,), jnp.int32),
        pltpu.VMEM((EPW,), jnp.int32),
        pltpu.VMEM((EPW,), jnp.float32),
        pltpu.VMEM((CK, D), jnp.float32),
        pltpu.VMEM((CK, D), jnp.float32),
        pltpu.VMEM_SHARED((N, D), jnp.float32),
        pltpu.SemaphoreType.DMA,
        pltpu.SemaphoreType.DMA,
        pltpu.SemaphoreType.DMA,
    ],
)
def _spmm_kernel(feat_hbm, src_hbm, dst_hbm, w_hbm, out_hbm,
                 src_v, dst_v, w_v, rowsA, rowsB, acc_sh, gsA, gsB, tsA):
    cid = lax.axis_index("c")
    sid = lax.axis_index("s")
    wid = sid * NC + cid
    ebase = wid * EPW

    pltpu.sync_copy(src_hbm.at[pl.ds(ebase, EPW)], src_v)
    pltpu.sync_copy(dst_hbm.at[pl.ds(ebase, EPW)], dst_v)
    pltpu.sync_copy(w_hbm.at[pl.ds(ebase, EPW)], w_v)

    def zbuf_row(i, _):
        for r in range(D // K):
            rowsA[i, pl.ds(r * K, K)] = jnp.zeros((K,), jnp.float32)
        return 0

    lax.fori_loop(0, CK, zbuf_row, 0)
    nflush = RPF // CK

    @pl.when(sid < NS - 1)
    def _():
        for j in range(nflush):
            pltpu.sync_copy(rowsA, acc_sh.at[pl.ds(sid * RPF + j * CK, CK)])

    @pl.when(sid == NS - 1)
    def _():
        for j in range((N - (NS - 1) * RPF) // CK):
            pltpu.sync_copy(rowsA,
                            acc_sh.at[pl.ds((NS - 1) * RPF + j * CK, CK)])

    plsc.subcore_barrier()

    pltpu.async_copy(feat_hbm.at[src_v.at[pl.ds(0, CK)]], rowsA, gsA)
    pltpu.async_copy(feat_hbm.at[src_v.at[pl.ds(CK, CK)]], rowsB, gsB)

    def chunk_body(g, _):
        def process(buf, sem):
            pltpu.make_async_copy(
                feat_hbm.at[src_v.at[pl.ds(g * CK, CK)]], buf, sem).wait()
            descs = []
            for q in range(QG):
                w16 = w_v[pl.ds(g * CK + q * K, K)]
                for k in range(K):
                    wb = w16.at[jnp.full((K,), k, jnp.int32)].get(
                        mode="promise_in_bounds")
                    e = q * K + k
                    for r in range(D // K):
                        sl = pl.ds(r * K, K)
                        buf[e, sl] = buf[e, sl] * wb
                d16 = dst_v[pl.ds(g * CK + q * K, K)]
                descs.append(pltpu.async_copy(
                    buf.at[pl.ds(q * K, K)], acc_sh.at[d16], tsA, add=True))
            for dsc in descs:
                dsc.wait()

            @pl.when(g + 2 < CC)
            def _():
                pltpu.async_copy(
                    feat_hbm.at[src_v.at[pl.ds((g + 2) * CK, CK)]], buf, sem)

        @pl.when(g % 2 == 0)
        def _():
            process(rowsA, gsA)

        @pl.when(g % 2 == 1)
        def _():
            process(rowsB, gsB)

        return 0

    lax.fori_loop(0, CC, chunk_body, 0)
    plsc.subcore_barrier()

    @pl.when(sid < NS - 1)
    def _():
        for j in range(nflush):
            rs = sid * RPF + j * CK
            pltpu.sync_copy(acc_sh.at[pl.ds(rs, CK)], rowsA)
            pltpu.sync_copy(rowsA, out_hbm.at[cid, pl.ds(rs, CK)])

    @pl.when(sid == NS - 1)
    def _():
        for j in range((N - (NS - 1) * RPF) // CK):
            rs = (NS - 1) * RPF + j * CK
            pltpu.sync_copy(acc_sh.at[pl.ds(rs, CK)], rowsA)
            pltpu.sync_copy(rowsA, out_hbm.at[cid, pl.ds(rs, CK)])


def _prep_body(dp_ref, x_ref, alpha_ref, beta_ref, feat_ref):
    s = jnp.sum(dp_ref[...], axis=0)
    wdeg_src = s[0]
    wdeg_dst = s[1]
    out_deg = jnp.maximum(s[2], 1.0)
    in_deg = jnp.maximum(s[3], 1.0)
    alpha = lax.rsqrt(wdeg_src * out_deg)
    beta = lax.rsqrt(wdeg_dst * in_deg)
    alpha2 = alpha[:, None]
    beta2 = beta[:, None]
    alpha_ref[...] = alpha2
    beta_ref[...] = beta2
    feat_ref[...] = x_ref[...] * alpha2


def _prep(deg_parts, x):
    return pl.pallas_call(
        _prep_body,
        out_shape=(
            jax.ShapeDtypeStruct((N, 1), jnp.float32),
            jax.ShapeDtypeStruct((N, 1), jnp.float32),
            jax.ShapeDtypeStruct((N, D), jnp.float32),
        ),
    )(deg_parts, x)


_RB = 1000


def _stage_body(last, p0_ref, p1_ref, beta_ref, W_ref, b_ref, alpha_ref,
                *out_refs):
    agg = (p0_ref[...] + p1_ref[...]) * beta_ref[...]
    z = jnp.dot(agg, W_ref[...], preferred_element_type=jnp.float32)
    z = z + b_ref[...]
    zn = jnp.minimum(z, 0.0)
    poly = zn * (1.0 + zn * (0.5 + zn * (1.0 / 6.0 + zn * (1.0 / 24.0))))
    em1 = jnp.where(zn > -0.05, poly, jnp.exp(zn) - 1.0)
    h = _SELU_L * jnp.where(z > 0, z, _SELU_A * em1)
    if last:
        out_refs[0][...] = h
    else:
        out_refs[0][...] = h * alpha_ref[...]


def _stage(p0, p1, beta, W, b, alpha, last):
    grid = N // _RB
    return pl.pallas_call(
        functools.partial(_stage_body, last),
        grid=(grid,),
        in_specs=[
            pl.BlockSpec((_RB, D), lambda i: (i, 0)),
            pl.BlockSpec((_RB, D), lambda i: (i, 0)),
            pl.BlockSpec((_RB, 1), lambda i: (i, 0)),
            pl.BlockSpec((D, D), lambda i: (0, 0)),
            pl.BlockSpec((1, D), lambda i: (0, 0)),
            pl.BlockSpec((_RB, 1), lambda i: (i, 0)),
        ],
        out_specs=pl.BlockSpec((_RB, D), lambda i: (i, 0)),
        out_shape=jax.ShapeDtypeStruct((N, D), jnp.float32),
    )(p0, p1, beta, W, b, alpha)


def kernel(x, edge_index, edge_weight, W1_0, b1_0, W2_0, b2_0,
           W1_1, b1_1, W2_1, b2_1):
    src_r = edge_index[0]
    dst_r = edge_index[1]
    w_r = edge_weight

    deg_parts = _deg_kernel(src_r, dst_r, w_r).reshape(NW, 4, N)
    alpha, beta, feat = _prep(deg_parts, x)

    weights = ((W1_0, b1_0), (W2_0, b2_0), (W1_1, b1_1), (W2_1, b2_1))
    for i, (W, b) in enumerate(weights):
        parts = _spmm_kernel(feat, src_r, dst_r, w_r)
        feat = _stage(parts[0], parts[1], beta, W, b.reshape(1, D), alpha,
                      last=(i == 3))
    return feat


# speedup vs baseline: 1.2839x; 1.0081x over previous
import functools

import jax
import jax.numpy as jnp
from jax import lax
from jax.experimental import pallas as pl
from jax.experimental.pallas import tpu as pltpu
from jax.experimental.pallas import tpu_sc as plsc

N = 10000
E = 320000
D = 128

NC = 2
NS = 16
NW = NC * NS
EPW = E // NW
K = 16
C = EPW // K
N_PAD = 10240
RPT = N_PAD // NS
FR = 64

_MESH = plsc.VectorSubcoreMesh(core_axis_name="c", subcore_axis_name="s")

_SELU_L = 1.0507009873554804934193349852946
_SELU_A = 1.6732632423543772848170429916717


@functools.partial(
    pl.kernel,
    out_type=jax.ShapeDtypeStruct((NW * 4 * N,), jnp.float32),
    mesh=_MESH,
    compiler_params=pltpu.CompilerParams(needs_layout_passes=False),
    scratch_types=[
        pltpu.VMEM((EPW,), jnp.int32),
        pltpu.VMEM((EPW,), jnp.int32),
        pltpu.VMEM((EPW,), jnp.float32),
        pltpu.VMEM((4 * N,), jnp.float32),
    ],
)
def _deg_kernel(src_hbm, dst_hbm, w_hbm, out_hbm, src_v, dst_v, w_v, acc_v):
    cid = lax.axis_index("c")
    sid = lax.axis_index("s")
    wid = sid * NC + cid

    pltpu.sync_copy(src_hbm.at[pl.ds(wid * EPW, EPW)], src_v)
    pltpu.sync_copy(dst_hbm.at[pl.ds(wid * EPW, EPW)], dst_v)
    pltpu.sync_copy(w_hbm.at[pl.ds(wid * EPW, EPW)], w_v)

    def zero_body(i, _):
        acc_v[pl.ds(i * K, K)] = jnp.zeros((K,), jnp.float32)
        return 0

    lax.fori_loop(0, (4 * N) // K, zero_body, 0)

    ones = jnp.ones((K,), jnp.float32)

    def edge_body(g, _):
        s16 = src_v[pl.ds(g * K, K)]
        d16 = dst_v[pl.ds(g * K, K)]
        w16 = w_v[pl.ds(g * K, K)]
        plsc.addupdate_scatter(acc_v, [s16], w16)
        plsc.addupdate_scatter(acc_v, [d16 + N], w16)
        plsc.addupdate_scatter(acc_v, [s16 + 2 * N], ones)
        plsc.addupdate_scatter(acc_v, [d16 + 3 * N], ones)
        return 0

    lax.fori_loop(0, C, edge_body, 0)

    pltpu.sync_copy(acc_v, out_hbm.at[pl.ds(wid * 4 * N, 4 * N)])


CK = 80
CC = EPW // CK
QG = CK // K
RPF = 640


def _spmm_scale(rows_b, w_v, g):
    for q in range(QG):
        w16 = w_v[pl.ds(g * CK + q * K, K)]
        for k in range(K):
            wb = w16.at[jnp.full((K,), k, jnp.int32)].get(
                mode="promise_in_bounds")
            e = q * K + k
            for r in range(D // K):
                sl = pl.ds(r * K, K)
                rows_b[e, sl] = rows_b[e, sl] * wb


def _spmm_scatter(rows_b, dst_v, acc_sh, g, sem):
    descs = []
    for q in range(QG):
        d16 = dst_v[pl.ds(g * CK + q * K, K)]
        descs.append(pltpu.async_copy(
            rows_b.at[pl.ds(q * K, K)], acc_sh.at[d16], sem, add=True))
    return descs


@functools.partial(
    pl.kernel,
    out_type=jax.ShapeDtypeStruct((NC, N, D), jnp.float32),
    mesh=_MESH,
    compiler_params=pltpu.CompilerParams(needs_layout_passes=False),
    scratch_types=[
        pltpu.VMEM((EPW,), jnp.int32),
        pltpu.VMEM((EPW,), jnp.int32),
        pltpu.VMEM((EPW,), jnp.float32),
        pltpu.VMEM((CK, D), jnp.float32),
        pltpu.VMEM((CK, D), jnp.float32),
        pltpu.VMEM_SHARED((N, D), jnp.float32),
        pltpu.SemaphoreType.DMA,
        pltpu.SemaphoreType.DMA,
        pltpu.SemaphoreType.DMA,
    ],
)
def _spmm_kernel(feat_hbm, src_hbm, dst_hbm, w_hbm, out_hbm,
                 src_v, dst_v, w_v, rowsA, rowsB, acc_sh, gsA, gsB, tsA):
    cid = lax.axis_index("c")
    sid = lax.axis_index("s")
    wid = sid * NC + cid
    ebase = wid * EPW

    pltpu.sync_copy(src_hbm.at[pl.ds(ebase, EPW)], src_v)
    pltpu.sync_copy(dst_hbm.at[pl.ds(ebase, EPW)], dst_v)
    pltpu.sync_copy(w_hbm.at[pl.ds(ebase, EPW)], w_v)

    def zbuf_row(i, _):
        for r in range(D // K):
            rowsA[i, pl.ds(r * K, K)] = jnp.zeros((K,), jnp.float32)
        return 0

    lax.fori_loop(0, CK, zbuf_row, 0)
    nflush = RPF // CK

    @pl.when(sid < NS - 1)
    def _():
        for j in range(nflush):
            pltpu.sync_copy(rowsA, acc_sh.at[pl.ds(sid * RPF + j * CK, CK)])

    @pl.when(sid == NS - 1)
    def _():
        for j in range((N - (NS - 1) * RPF) // CK):
            pltpu.sync_copy(rowsA,
                            acc_sh.at[pl.ds((NS - 1) * RPF + j * CK, CK)])

    plsc.subcore_barrier()

    pltpu.async_copy(feat_hbm.at[src_v.at[pl.ds(0, CK)]], rowsA, gsA)
    pltpu.async_copy(feat_hbm.at[src_v.at[pl.ds(CK, CK)]], rowsB, gsB)

    def chunk_body(g, _):
        def process(buf, sem):
            pltpu.make_async_copy(
                feat_hbm.at[src_v.at[pl.ds(g * CK, CK)]], buf, sem).wait()
            descs = []
            for q in range(QG):
                w16 = w_v[pl.ds(g * CK + q * K, K)]
                for k in range(K):
                    wb = w16.at[jnp.full((K,), k, jnp.int32)].get(
                        mode="promise_in_bounds")
                    e = q * K + k
                    for r in range(D // K):
                        sl = pl.ds(r * K, K)
                        buf[e, sl] = buf[e, sl] * wb
                d16 = dst_v[pl.ds(g * CK + q * K, K)]
                descs.append(pltpu.async_copy(
                    buf.at[pl.ds(q * K, K)], acc_sh.at[d16], tsA, add=True))
            for dsc in descs:
                dsc.wait()

            @pl.when(g + 2 < CC)
            def _():
                pltpu.async_copy(
                    feat_hbm.at[src_v.at[pl.ds((g + 2) * CK, CK)]], buf, sem)

        @pl.when(g % 2 == 0)
        def _():
            process(rowsA, gsA)

        @pl.when(g % 2 == 1)
        def _():
            process(rowsB, gsB)

        return 0

    lax.fori_loop(0, CC, chunk_body, 0)
    plsc.subcore_barrier()

    @pl.when(sid < NS - 1)
    def _():
        pltpu.sync_copy(acc_sh.at[pl.ds(sid * RPF, RPF)],
                        out_hbm.at[cid, pl.ds(sid * RPF, RPF)])

    @pl.when(sid == NS - 1)
    def _():
        pltpu.sync_copy(acc_sh.at[pl.ds((NS - 1) * RPF, N - (NS - 1) * RPF)],
                        out_hbm.at[cid, pl.ds((NS - 1) * RPF,
                                              N - (NS - 1) * RPF)])


def _prep_body(dp_ref, x_ref, alpha_ref, beta_ref, feat_ref):
    s = jnp.sum(dp_ref[...], axis=0)
    wdeg_src = s[0]
    wdeg_dst = s[1]
    out_deg = jnp.maximum(s[2], 1.0)
    in_deg = jnp.maximum(s[3], 1.0)
    alpha = lax.rsqrt(wdeg_src * out_deg)
    beta = lax.rsqrt(wdeg_dst * in_deg)
    alpha2 = alpha[:, None]
    beta2 = beta[:, None]
    alpha_ref[...] = alpha2
    beta_ref[...] = beta2
    feat_ref[...] = x_ref[...] * alpha2


def _prep(deg_parts, x):
    return pl.pallas_call(
        _prep_body,
        out_shape=(
            jax.ShapeDtypeStruct((N, 1), jnp.float32),
            jax.ShapeDtypeStruct((N, 1), jnp.float32),
            jax.ShapeDtypeStruct((N, D), jnp.float32),
        ),
    )(deg_parts, x)


_RB = 1000


def _stage_body(last, p0_ref, p1_ref, beta_ref, W_ref, b_ref, alpha_ref,
                *out_refs):
    agg = (p0_ref[...] + p1_ref[...]) * beta_ref[...]
    z = jnp.dot(agg, W_ref[...], preferred_element_type=jnp.float32)
    z = z + b_ref[...]
    zn = jnp.minimum(z, 0.0)
    poly = zn * (1.0 + zn * (0.5 + zn * (1.0 / 6.0 + zn * (1.0 / 24.0))))
    em1 = jnp.where(zn > -0.05, poly, jnp.exp(zn) - 1.0)
    h = _SELU_L * jnp.where(z > 0, z, _SELU_A * em1)
    if last:
        out_refs[0][...] = h
    else:
        out_refs[0][...] = h * alpha_ref[...]


def _stage(p0, p1, beta, W, b, alpha, last):
    grid = N // _RB
    return pl.pallas_call(
        functools.partial(_stage_body, last),
        grid=(grid,),
        in_specs=[
            pl.BlockSpec((_RB, D), lambda i: (i, 0)),
            pl.BlockSpec((_RB, D), lambda i: (i, 0)),
            pl.BlockSpec((_RB, 1), lambda i: (i, 0)),
            pl.BlockSpec((D, D), lambda i: (0, 0)),
            pl.BlockSpec((1, D), lambda i: (0, 0)),
            pl.BlockSpec((_RB, 1), lambda i: (i, 0)),
        ],
        out_specs=pl.BlockSpec((_RB, D), lambda i: (i, 0)),
        out_shape=jax.ShapeDtypeStruct((N, D), jnp.float32),
    )(p0, p1, beta, W, b, alpha)


def kernel(x, edge_index, edge_weight, W1_0, b1_0, W2_0, b2_0,
           W1_1, b1_1, W2_1, b2_1):
    src_r = edge_index[0]
    dst_r = edge_index[1]
    w_r = edge_weight

    deg_parts = _deg_kernel(src_r, dst_r, w_r).reshape(NW, 4, N)
    alpha, beta, feat = _prep(deg_parts, x)

    weights = ((W1_0, b1_0), (W2_0, b2_0), (W1_1, b1_1), (W2_1, b2_1))
    for i, (W, b) in enumerate(weights):
        parts = _spmm_kernel(feat, src_r, dst_r, w_r)
        feat = _stage(parts[0], parts[1], beta, W, b.reshape(1, D), alpha,
                      last=(i == 3))
    return feat
